# baseline (speedup 1.0000x reference)
import functools

import jax
import jax.numpy as jnp
from jax import lax
from jax.experimental import pallas as pl
from jax.experimental.pallas import tpu as pltpu
from jax.experimental.pallas import tpu_sc as plsc

VOCAB = 100000
D = 64
BATCH = 16384
HIST = 50

NC = 2
NS = 16
NW = NC * NS

_mesh = plsc.VectorSubcoreMesh(core_axis_name="c", subcore_axis_name="s")


@functools.partial(
    pl.kernel,
    mesh=_mesh,
    out_type=jax.ShapeDtypeStruct((HIST, VOCAB), jnp.float32),
    scratch_types=[
        pltpu.VMEM((BATCH,), jnp.int32),
        pltpu.VMEM((VOCAB,), jnp.float32),
        pltpu.SemaphoreType.DMA,
    ],
    compiler_params=pltpu.CompilerParams(
        use_tc_tiling_on_sc=False, needs_layout_passes=False
    ),
)
def _sc_hist(idxT_hbm, out_hbm, idxv, cnt, sem):
    wid = lax.axis_index("s") * NC + lax.axis_index("c")
    zero = jnp.zeros((16,), jnp.float32)
    ones = jnp.ones((16,), jnp.float32)

    for p in range(2):
        l = p * NW + wid

        @pl.when(l < HIST)
        def _():
            pltpu.sync_copy(idxT_hbm.at[l], idxv)

            def zbody(i, carry):
                base = i * 160
                for u in range(10):
                    cnt[pl.ds(base + u * 16, 16)] = zero
                return carry

            lax.fori_loop(0, VOCAB // 160, zbody, 0)

            def sbody(i, carry):
                base = i * 128
                for u in range(8):
                    iv = idxv[pl.ds(base + u * 16, 16)]
                    plsc.addupdate_scatter(cnt, [iv], ones)
                return carry

            lax.fori_loop(0, BATCH // 128, sbody, 0)

            pltpu.sync_copy(cnt, out_hbm.at[l])


VK = 8192
NK = VOCAB // VK
TAIL = VOCAB - NK * VK


def _mean_body(cnt_ref, t_ref, ctail_ref, ttail_ref, o_ref):
    k = pl.program_id(0)

    @pl.when(k == 0)
    def _():
        o_ref[...] = lax.dot_general(
            ctail_ref[...], ttail_ref[...], (((1,), (0,)), ((), ())),
            preferred_element_type=jnp.float32,
        )

    o_ref[...] += lax.dot_general(
        cnt_ref[...], t_ref[...], (((1,), (0,)), ((), ())),
        preferred_element_type=jnp.float32,
    )

    @pl.when(k == NK - 1)
    def _():
        o_ref[...] *= 1.0 / BATCH


_mean = pl.pallas_call(
    _mean_body,
    grid=(NK,),
    in_specs=[
        pl.BlockSpec((HIST, VK), lambda k: (0, k)),
        pl.BlockSpec((VK, D), lambda k: (k, 0)),
        pl.BlockSpec((HIST, TAIL), lambda k: (0, 0)),
        pl.BlockSpec((TAIL, D), lambda k: (0, 0)),
    ],
    out_specs=pl.BlockSpec((HIST, D), lambda k: (0, 0)),
    out_shape=jax.ShapeDtypeStruct((HIST, D), jnp.float32),
)


VC = 8192


def _mm_body(mean_ref, w_ref, b_ref, o_ref):
    o_ref[...] = (
        lax.dot_general(
            mean_ref[...], w_ref[...], (((1,), (1,)), ((), ())),
            preferred_element_type=jnp.float32,
        )
        + b_ref[...]
    )


_project = pl.pallas_call(
    _mm_body,
    grid=(pl.cdiv(VOCAB, VC),),
    in_specs=[
        pl.BlockSpec((HIST, D), lambda j: (0, 0)),
        pl.BlockSpec((VC, D), lambda j: (j, 0)),
        pl.BlockSpec((1, VC), lambda j: (0, j)),
    ],
    out_specs=pl.BlockSpec((HIST, VC), lambda j: (0, j)),
    out_shape=jax.ShapeDtypeStruct((HIST, VOCAB), jnp.float32),
)


def kernel(context_idxs, emb_table, W, b):
    idxT = context_idxs.astype(jnp.int32).T
    counts = _sc_hist(idxT)
    mean = _mean(
        counts,
        emb_table,
        counts[:, NK * VK :],
        emb_table[NK * VK :],
    )
    return _project(mean, W, b.reshape(1, VOCAB))


# speedup vs baseline: 1.0233x; 1.0233x over previous
import functools

import jax
import jax.numpy as jnp
from jax import lax
from jax.experimental import pallas as pl
from jax.experimental.pallas import tpu as pltpu
from jax.experimental.pallas import tpu_sc as plsc

VOCAB = 100000
D = 64
BATCH = 16384
HIST = 50

NC = 2
NS = 16
NW = NC * NS

_mesh = plsc.VectorSubcoreMesh(core_axis_name="c", subcore_axis_name="s")


HALF = VOCAB // 2


@functools.partial(
    pl.kernel,
    mesh=_mesh,
    out_type=jax.ShapeDtypeStruct((HIST, HALF), jnp.int32),
    scratch_types=[
        pltpu.VMEM((BATCH,), jnp.int32),
        pltpu.VMEM((VOCAB,), jnp.int32),
        pltpu.SemaphoreType.DMA,
    ],
    compiler_params=pltpu.CompilerParams(
        use_tc_tiling_on_sc=False, needs_layout_passes=False
    ),
)
def _sc_hist(idxT_hbm, out_hbm, idxv, cnt, sem):
    wid = lax.axis_index("s") * NC + lax.axis_index("c")
    zero = jnp.zeros((16,), jnp.int32)
    ones = jnp.ones((16,), jnp.int32)

    for p in range(2):
        l = p * NW + wid

        @pl.when(l < HIST)
        def _():
            pltpu.sync_copy(idxT_hbm.at[l], idxv)

            def zbody(i, carry):
                base = i * 160
                for u in range(10):
                    cnt[pl.ds(base + u * 16, 16)] = zero
                return carry

            lax.fori_loop(0, VOCAB // 160, zbody, 0)

            def sbody(i, carry):
                base = i * 128
                for u in range(8):
                    iv = idxv[pl.ds(base + u * 16, 16)]
                    plsc.addupdate_scatter(cnt, [iv], ones)
                return carry

            lax.fori_loop(0, BATCH // 128, sbody, 0)

            def pbody(i, carry):
                base = i * 80
                for u in range(5):
                    j = base + u * 16
                    a = cnt[pl.ds(j, 16)]
                    b = cnt[pl.ds(HALF + j, 16)]
                    cnt[pl.ds(j, 16)] = a + b * 65536
                return carry

            lax.fori_loop(0, HALF // 80, pbody, 0)

            pltpu.sync_copy(cnt.at[pl.ds(0, HALF)], out_hbm.at[l])


VK = 8192
NK = HALF // VK
TAIL = HALF - NK * VK


def _unpack(x):
    lo = (x & 65535).astype(jnp.float32)
    hi = lax.shift_right_logical(x, 16).astype(jnp.float32)
    return lo, hi


def _mean_body(pk_ref, tlo_ref, thi_ref, pkt_ref, tlot_ref, thit_ref, o_ref):
    k = pl.program_id(0)

    def _dot(a, b):
        return lax.dot_general(
            a, b, (((1,), (0,)), ((), ())), preferred_element_type=jnp.float32
        )

    @pl.when(k == 0)
    def _():
        lo_t, hi_t = _unpack(pkt_ref[...])
        o_ref[...] = _dot(lo_t, tlot_ref[...]) + _dot(hi_t, thit_ref[...])

    lo, hi = _unpack(pk_ref[...])
    o_ref[...] += _dot(lo, tlo_ref[...]) + _dot(hi, thi_ref[...])

    @pl.when(k == NK - 1)
    def _():
        o_ref[...] *= 1.0 / BATCH


_mean = pl.pallas_call(
    _mean_body,
    grid=(NK,),
    in_specs=[
        pl.BlockSpec((HIST, VK), lambda k: (0, k)),
        pl.BlockSpec((VK, D), lambda k: (k, 0)),
        pl.BlockSpec((VK, D), lambda k: (k, 0)),
        pl.BlockSpec((HIST, TAIL), lambda k: (0, 0)),
        pl.BlockSpec((TAIL, D), lambda k: (0, 0)),
        pl.BlockSpec((TAIL, D), lambda k: (0, 0)),
    ],
    out_specs=pl.BlockSpec((HIST, D), lambda k: (0, 0)),
    out_shape=jax.ShapeDtypeStruct((HIST, D), jnp.float32),
)


VC = 8192


def _mm_body(mean_ref, w_ref, b_ref, o_ref):
    o_ref[...] = (
        lax.dot_general(
            mean_ref[...], w_ref[...], (((1,), (1,)), ((), ())),
            preferred_element_type=jnp.float32,
        )
        + b_ref[...]
    )


_project = pl.pallas_call(
    _mm_body,
    grid=(pl.cdiv(VOCAB, VC),),
    in_specs=[
        pl.BlockSpec((HIST, D), lambda j: (0, 0)),
        pl.BlockSpec((VC, D), lambda j: (j, 0)),
        pl.BlockSpec((1, VC), lambda j: (0, j)),
    ],
    out_specs=pl.BlockSpec((HIST, VC), lambda j: (0, j)),
    out_shape=jax.ShapeDtypeStruct((HIST, VOCAB), jnp.float32),
)


def kernel(context_idxs, emb_table, W, b):
    idxT = context_idxs.astype(jnp.int32).T
    packed = _sc_hist(idxT)
    mean = _mean(
        packed,
        emb_table,
        emb_table[HALF:],
        packed[:, NK * VK :],
        emb_table[NK * VK : HALF],
        emb_table[HALF + NK * VK :],
    )
    return _project(mean, W, b.reshape(1, VOCAB))
